# baseline (speedup 1.0000x reference)
import functools

import jax
import jax.numpy as jnp
from jax import lax
from jax.experimental import pallas as pl
from jax.experimental.pallas import tpu as pltpu
from jax.experimental.pallas import tpu_sc as plsc

N = 10000
E = 320000
D = 128
NC = 2
NS = 16
NW = NC * NS
EW = E // NW
B = 125
NB = EW // B
CH = 40
NCH = NB // CH
RPT = N // NS


def _sc_mesh():
    return plsc.VectorSubcoreMesh(core_axis_name="c", subcore_axis_name="s")


_SC_PARAMS = pltpu.CompilerParams(use_tc_tiling_on_sc=False)


def _deg_kernel(ei_hbm, ones_hbm, z16_hbm, out_hbm, idx_f, idx_b, ones_v,
                acc_f, acc_b):
    c = lax.axis_index("c")
    s = lax.axis_index("s")
    w = s * NC + c
    pltpu.sync_copy(z16_hbm.at[pl.ds(s * RPT, RPT)],
                    acc_f.at[pl.ds(s * RPT, RPT)])
    pltpu.sync_copy(z16_hbm.at[pl.ds(s * RPT, RPT)],
                    acc_b.at[pl.ds(s * RPT, RPT)])
    pltpu.sync_copy(ones_hbm, ones_v)
    pltpu.sync_copy(ei_hbm.at[1, w], idx_f)
    pltpu.sync_copy(ei_hbm.at[0, w], idx_b)
    plsc.subcore_barrier()

    def body(j, carry):
        pltpu.sync_copy(ones_v, acc_f.at[idx_f.at[j]], add=True)
        pltpu.sync_copy(ones_v, acc_b.at[idx_b.at[j]], add=True)
        return carry

    lax.fori_loop(0, NB, body, 0)
    plsc.subcore_barrier()
    pltpu.sync_copy(acc_f.at[pl.ds(s * RPT, RPT), pl.ds(0, 8)],
                    out_hbm.at[0, c, pl.ds(s * RPT, RPT)])
    pltpu.sync_copy(acc_b.at[pl.ds(s * RPT, RPT), pl.ds(0, 8)],
                    out_hbm.at[1, c, pl.ds(s * RPT, RPT)])


def _degrees(ei, ones16, zeros16):
    k = pl.kernel(
        _deg_kernel,
        out_type=jax.ShapeDtypeStruct((2, NC, N, 8), jnp.float32),
        mesh=_sc_mesh(),
        compiler_params=_SC_PARAMS,
        scratch_types=[
            pltpu.VMEM((NB, B), jnp.int32),
            pltpu.VMEM((NB, B), jnp.int32),
            pltpu.VMEM((B, 16), jnp.float32),
            pltpu.VMEM_SHARED((N, 16), jnp.float32),
            pltpu.VMEM_SHARED((N, 16), jnp.float32),
        ],
    )
    return k(ei, ones16, zeros16)


def _prop_kernel(ei_hbm, zf_hbm, zb_hbm, z128_hbm, outf_hbm, outb_hbm,
                 gidx, sidx, rows0, rows1, sg0, sg1, acc):
    c = lax.axis_index("c")
    s = lax.axis_index("s")
    w = s * NC + c
    for d, z_hbm, out_hbm in ((0, zf_hbm, outf_hbm), (1, zb_hbm, outb_hbm)):
        @pl.when(c == 0)
        def _():
            pltpu.sync_copy(z_hbm.at[pl.ds(s * RPT, RPT)],
                            acc.at[pl.ds(s * RPT, RPT)])

        @pl.when(c == 1)
        def _():
            pltpu.sync_copy(z128_hbm.at[pl.ds(s * RPT, RPT)],
                            acc.at[pl.ds(s * RPT, RPT)])

        plsc.subcore_barrier()

        for ch in range(NCH):
            pltpu.sync_copy(ei_hbm.at[d, w, pl.ds(ch * CH, CH)], gidx)
            pltpu.sync_copy(ei_hbm.at[1 - d, w, pl.ds(ch * CH, CH)], sidx)
            pltpu.async_copy(z_hbm.at[gidx.at[0]], rows0, sg0)

            def body(i, carry):
                b0 = 2 * i
                b1 = b0 + 1
                pltpu.make_async_copy(z_hbm.at[gidx.at[b0]], rows0,
                                      sg0).wait()
                pltpu.async_copy(z_hbm.at[gidx.at[b1]], rows1, sg1)
                pltpu.sync_copy(rows0, acc.at[sidx.at[b0]], add=True)
                nxt = lax.min(b0 + 2, CH - 1)
                pltpu.make_async_copy(z_hbm.at[gidx.at[b1]], rows1,
                                      sg1).wait()
                pltpu.async_copy(z_hbm.at[gidx.at[nxt]], rows0, sg0)
                pltpu.sync_copy(rows1, acc.at[sidx.at[b1]], add=True)
                return carry

            lax.fori_loop(0, CH // 2, body, 0)
            pltpu.make_async_copy(z_hbm.at[gidx.at[0]], rows0, sg0).wait()
        plsc.subcore_barrier()
        pltpu.sync_copy(acc.at[pl.ds(s * RPT, RPT)],
                        out_hbm.at[c, pl.ds(s * RPT, RPT)])
        plsc.subcore_barrier()


def _propagate(ei, zf, zb, zeros128):
    k = pl.kernel(
        _prop_kernel,
        out_type=(
            jax.ShapeDtypeStruct((NC, N, D), jnp.float32),
            jax.ShapeDtypeStruct((NC, N, D), jnp.float32),
        ),
        mesh=_sc_mesh(),
        compiler_params=_SC_PARAMS,
        scratch_types=[
            pltpu.VMEM((CH, B), jnp.int32),
            pltpu.VMEM((CH, B), jnp.int32),
            pltpu.VMEM((B, D), jnp.float32),
            pltpu.VMEM((B, D), jnp.float32),
            pltpu.SemaphoreType.DMA,
            pltpu.SemaphoreType.DMA,
            pltpu.VMEM_SHARED((N, D), jnp.float32),
        ],
    )
    return k(ei, zf, zb, zeros128)


_R = N


def _tc1_body(x_r, w1f_r, w1b_r, df_r, db_r, z1f_r, z1b_r):
    df = df_r[...]
    db = db_r[...]
    xv = x_r[...]
    z1f_r[...] = df * jnp.dot(xv, w1f_r[...], preferred_element_type=jnp.float32)
    z1b_r[...] = db * jnp.dot(xv, w1b_r[...], preferred_element_type=jnp.float32)


def _tc2_body(pf_r, b1f_r, w2f_r, df_r, pb_r, b1b_r, w2b_r,
              db_r, z2f_r, z2b_r):
    df = df_r[...]
    db = db_r[...]
    hf = jnp.maximum(df * (pf_r[0] + pf_r[1]) + b1f_r[...], 0.0)
    hb = jnp.maximum(db * (pb_r[0] + pb_r[1]) + b1b_r[...], 0.0)
    z2f_r[...] = df * jnp.dot(hf, w2f_r[...], preferred_element_type=jnp.float32)
    z2b_r[...] = db * jnp.dot(hb, w2b_r[...], preferred_element_type=jnp.float32)


def _tc3_body(pf_r, b2f_r, wfa_r, df_r, pb_r, b2b_r, wfb_r,
              db_r, bfin_r, out_r):
    df = df_r[...]
    db = db_r[...]
    xf = df * (pf_r[0] + pf_r[1]) + b2f_r[...]
    xb = db * (pb_r[0] + pb_r[1]) + b2b_r[...]
    out_r[...] = (jnp.dot(xf, wfa_r[...], preferred_element_type=jnp.float32)
                  + jnp.dot(xb, wfb_r[...], preferred_element_type=jnp.float32)
                  + bfin_r[...])


def _nd(i):
    return (i, 0)


_SPEC_ROWS = pl.BlockSpec((_R, D), _nd)
_SPEC_W = pl.BlockSpec((D, D), lambda i: (0, 0))
_SPEC_B = pl.BlockSpec((1, D), lambda i: (0, 0))
_SPEC_DINV = pl.BlockSpec((_R, 1), lambda i: (i, 0))
_SPEC_P = pl.BlockSpec((NC, _R, D), lambda i: (0, i, 0))
_GRID = (N // _R,)


def _tc1(x, w1f, w1b, degf, degb):
    return pl.pallas_call(
        _tc1_body,
        grid=_GRID,
        in_specs=[_SPEC_ROWS, _SPEC_W, _SPEC_W, _SPEC_DINV, _SPEC_DINV],
        out_specs=[_SPEC_ROWS, _SPEC_ROWS],
        out_shape=[jax.ShapeDtypeStruct((N, D), jnp.float32)] * 2,
    )(x, w1f, w1b, degf, degb)


def _tc2(pf, b1f, w2f, degf, pb, b1b, w2b, degb):
    return pl.pallas_call(
        _tc2_body,
        grid=_GRID,
        in_specs=[_SPEC_P, _SPEC_B, _SPEC_W, _SPEC_DINV,
                  _SPEC_P, _SPEC_B, _SPEC_W, _SPEC_DINV],
        out_specs=[_SPEC_ROWS, _SPEC_ROWS],
        out_shape=[jax.ShapeDtypeStruct((N, D), jnp.float32)] * 2,
    )(pf, b1f, w2f, degf, pb, b1b, w2b, degb)


def _tc3(pf, b2f, wfa, degf, pb, b2b, wfb, degb, bfin):
    return pl.pallas_call(
        _tc3_body,
        grid=_GRID,
        in_specs=[_SPEC_P, _SPEC_B, _SPEC_W, _SPEC_DINV,
                  _SPEC_P, _SPEC_B, _SPEC_W, _SPEC_DINV, _SPEC_B],
        out_specs=_SPEC_ROWS,
        out_shape=jax.ShapeDtypeStruct((N, D), jnp.float32),
    )(pf, b2f, wfa, degf, pb, b2b, wfb, degb, bfin)


def kernel(x, W1f, b1f, W2f, b2f, W1b, b1b, W2b, b2b, Wfin, bfin, edge_index):
    ei = edge_index.reshape(2, NW, NB, B)
    ones16 = jnp.ones((B, 16), jnp.float32)
    zeros16 = jnp.zeros((N, 16), jnp.float32)
    zeros128 = jnp.zeros((N, D), jnp.float32)
    b1f2 = b1f.reshape(1, D)
    b1b2 = b1b.reshape(1, D)
    b2f2 = b2f.reshape(1, D)
    b2b2 = b2b.reshape(1, D)
    bfin2 = bfin.reshape(1, D)
    wfa = Wfin[:D]
    wfb = Wfin[D:]

    degp = _degrees(ei, ones16, zeros16)
    dinvf = lax.rsqrt(degp[0, 0, :, 0:1] + degp[0, 1, :, 0:1] + 1.0)
    dinvb = lax.rsqrt(degp[1, 0, :, 0:1] + degp[1, 1, :, 0:1] + 1.0)

    z1f, z1b = _tc1(x, W1f, W1b, dinvf, dinvb)
    p1f, p1b = _propagate(ei, z1f, z1b, zeros128)
    z2f, z2b = _tc2(p1f, b1f2, W2f, dinvf, p1b, b1b2, W2b, dinvb)
    p2f, p2b = _propagate(ei, z2f, z2b, zeros128)
    return _tc3(p2f, b2f2, wfa, dinvf, p2b, b2b2, wfb, dinvb, bfin2)


# speedup vs baseline: 1.0084x; 1.0084x over previous
import functools

import jax
import jax.numpy as jnp
from jax import lax
from jax.experimental import pallas as pl
from jax.experimental.pallas import tpu as pltpu
from jax.experimental.pallas import tpu_sc as plsc

N = 10000
E = 320000
D = 128
NC = 2
NS = 16
NW = NC * NS
EW = E // NW
B = 125
NB = EW // B
CH = 40
NCH = NB // CH
RPT = N // NS


def _sc_mesh():
    return plsc.VectorSubcoreMesh(core_axis_name="c", subcore_axis_name="s")


_SC_PARAMS = pltpu.CompilerParams(use_tc_tiling_on_sc=False)


def _deg_kernel(ei_hbm, ones_hbm, z16_hbm, out_hbm, idx_f, idx_b, ones_v,
                acc_f, acc_b):
    c = lax.axis_index("c")
    s = lax.axis_index("s")
    w = s * NC + c
    pltpu.sync_copy(z16_hbm.at[pl.ds(s * RPT, RPT)],
                    acc_f.at[pl.ds(s * RPT, RPT)])
    pltpu.sync_copy(z16_hbm.at[pl.ds(s * RPT, RPT)],
                    acc_b.at[pl.ds(s * RPT, RPT)])
    pltpu.sync_copy(ones_hbm, ones_v)
    pltpu.sync_copy(ei_hbm.at[1, w], idx_f)
    pltpu.sync_copy(ei_hbm.at[0, w], idx_b)
    plsc.subcore_barrier()

    def body(j, carry):
        pltpu.sync_copy(ones_v, acc_f.at[idx_f.at[j]], add=True)
        pltpu.sync_copy(ones_v, acc_b.at[idx_b.at[j]], add=True)
        return carry

    lax.fori_loop(0, NB, body, 0)
    plsc.subcore_barrier()
    pltpu.sync_copy(acc_f.at[pl.ds(s * RPT, RPT)],
                    out_hbm.at[0, c, pl.ds(s * RPT, RPT)])
    pltpu.sync_copy(acc_b.at[pl.ds(s * RPT, RPT)],
                    out_hbm.at[1, c, pl.ds(s * RPT, RPT)])


def _degrees(ei, ones16, zeros16):
    k = pl.kernel(
        _deg_kernel,
        out_type=jax.ShapeDtypeStruct((2, NC, N, 16), jnp.float32),
        mesh=_sc_mesh(),
        compiler_params=_SC_PARAMS,
        scratch_types=[
            pltpu.VMEM((NB, B), jnp.int32),
            pltpu.VMEM((NB, B), jnp.int32),
            pltpu.VMEM((B, 16), jnp.float32),
            pltpu.VMEM_SHARED((N, 16), jnp.float32),
            pltpu.VMEM_SHARED((N, 16), jnp.float32),
        ],
    )
    return k(ei, ones16, zeros16)


def _prop_kernel(ei_hbm, zf_hbm, zb_hbm, z128_hbm, outf_hbm, outb_hbm,
                 gidx, sidx, rows0, rows1, sg0, sg1, acc):
    c = lax.axis_index("c")
    s = lax.axis_index("s")
    w = s * NC + c
    for d, z_hbm, out_hbm in ((0, zf_hbm, outf_hbm), (1, zb_hbm, outb_hbm)):
        @pl.when(c == 0)
        def _():
            pltpu.sync_copy(z_hbm.at[pl.ds(s * RPT, RPT)],
                            acc.at[pl.ds(s * RPT, RPT)])

        @pl.when(c == 1)
        def _():
            pltpu.sync_copy(z128_hbm.at[pl.ds(s * RPT, RPT)],
                            acc.at[pl.ds(s * RPT, RPT)])

        plsc.subcore_barrier()

        for ch in range(NCH):
            pltpu.sync_copy(ei_hbm.at[d, w, pl.ds(ch * CH, CH)], gidx)
            pltpu.sync_copy(ei_hbm.at[1 - d, w, pl.ds(ch * CH, CH)], sidx)
            pltpu.async_copy(z_hbm.at[gidx.at[0]], rows0, sg0)

            def body(i, carry):
                b0 = 2 * i
                b1 = b0 + 1
                pltpu.make_async_copy(z_hbm.at[gidx.at[b0]], rows0,
                                      sg0).wait()
                pltpu.async_copy(z_hbm.at[gidx.at[b1]], rows1, sg1)
                pltpu.sync_copy(rows0, acc.at[sidx.at[b0]], add=True)
                nxt = lax.min(b0 + 2, CH - 1)
                pltpu.make_async_copy(z_hbm.at[gidx.at[b1]], rows1,
                                      sg1).wait()
                pltpu.async_copy(z_hbm.at[gidx.at[nxt]], rows0, sg0)
                pltpu.sync_copy(rows1, acc.at[sidx.at[b1]], add=True)
                return carry

            lax.fori_loop(0, CH // 2, body, 0)
            pltpu.make_async_copy(z_hbm.at[gidx.at[0]], rows0, sg0).wait()
        plsc.subcore_barrier()
        pltpu.sync_copy(acc.at[pl.ds(s * RPT, RPT)],
                        out_hbm.at[c, pl.ds(s * RPT, RPT)])
        plsc.subcore_barrier()


def _propagate(ei, zf, zb, zeros128):
    k = pl.kernel(
        _prop_kernel,
        out_type=(
            jax.ShapeDtypeStruct((NC, N, D), jnp.float32),
            jax.ShapeDtypeStruct((NC, N, D), jnp.float32),
        ),
        mesh=_sc_mesh(),
        compiler_params=_SC_PARAMS,
        scratch_types=[
            pltpu.VMEM((CH, B), jnp.int32),
            pltpu.VMEM((CH, B), jnp.int32),
            pltpu.VMEM((B, D), jnp.float32),
            pltpu.VMEM((B, D), jnp.float32),
            pltpu.SemaphoreType.DMA,
            pltpu.SemaphoreType.DMA,
            pltpu.VMEM_SHARED((N, D), jnp.float32),
        ],
    )
    return k(ei, zf, zb, zeros128)


_R = N


def _tc1_body(x_r, w1f_r, w1b_r, df_r, db_r, z1f_r, z1b_r):
    df = df_r[...]
    db = db_r[...]
    xv = x_r[...]
    z1f_r[...] = df * jnp.dot(xv, w1f_r[...], preferred_element_type=jnp.float32)
    z1b_r[...] = db * jnp.dot(xv, w1b_r[...], preferred_element_type=jnp.float32)


def _tc2_body(pf_r, b1f_r, w2f_r, df_r, pb_r, b1b_r, w2b_r,
              db_r, z2f_r, z2b_r):
    df = df_r[...]
    db = db_r[...]
    hf = jnp.maximum(df * (pf_r[0] + pf_r[1]) + b1f_r[...], 0.0)
    hb = jnp.maximum(db * (pb_r[0] + pb_r[1]) + b1b_r[...], 0.0)
    z2f_r[...] = df * jnp.dot(hf, w2f_r[...], preferred_element_type=jnp.float32)
    z2b_r[...] = db * jnp.dot(hb, w2b_r[...], preferred_element_type=jnp.float32)


def _tc3_body(pf_r, b2f_r, wfa_r, df_r, pb_r, b2b_r, wfb_r,
              db_r, bfin_r, out_r):
    df = df_r[...]
    db = db_r[...]
    xf = df * (pf_r[0] + pf_r[1]) + b2f_r[...]
    xb = db * (pb_r[0] + pb_r[1]) + b2b_r[...]
    out_r[...] = (jnp.dot(xf, wfa_r[...], preferred_element_type=jnp.float32)
                  + jnp.dot(xb, wfb_r[...], preferred_element_type=jnp.float32)
                  + bfin_r[...])


def _nd(i):
    return (i, 0)


_SPEC_ROWS = pl.BlockSpec((_R, D), _nd)
_SPEC_W = pl.BlockSpec((D, D), lambda i: (0, 0))
_SPEC_B = pl.BlockSpec((1, D), lambda i: (0, 0))
_SPEC_DINV = pl.BlockSpec((_R, D), lambda i: (i, 0))
_SPEC_P = pl.BlockSpec((NC, _R, D), lambda i: (0, i, 0))
_GRID = (N // _R,)


def _tc1(x, w1f, w1b, degf, degb):
    return pl.pallas_call(
        _tc1_body,
        grid=_GRID,
        in_specs=[_SPEC_ROWS, _SPEC_W, _SPEC_W, _SPEC_DINV, _SPEC_DINV],
        out_specs=[_SPEC_ROWS, _SPEC_ROWS],
        out_shape=[jax.ShapeDtypeStruct((N, D), jnp.float32)] * 2,
    )(x, w1f, w1b, degf, degb)


def _tc2(pf, b1f, w2f, degf, pb, b1b, w2b, degb):
    return pl.pallas_call(
        _tc2_body,
        grid=_GRID,
        in_specs=[_SPEC_P, _SPEC_B, _SPEC_W, _SPEC_DINV,
                  _SPEC_P, _SPEC_B, _SPEC_W, _SPEC_DINV],
        out_specs=[_SPEC_ROWS, _SPEC_ROWS],
        out_shape=[jax.ShapeDtypeStruct((N, D), jnp.float32)] * 2,
    )(pf, b1f, w2f, degf, pb, b1b, w2b, degb)


def _tc3(pf, b2f, wfa, degf, pb, b2b, wfb, degb, bfin):
    return pl.pallas_call(
        _tc3_body,
        grid=_GRID,
        in_specs=[_SPEC_P, _SPEC_B, _SPEC_W, _SPEC_DINV,
                  _SPEC_P, _SPEC_B, _SPEC_W, _SPEC_DINV, _SPEC_B],
        out_specs=_SPEC_ROWS,
        out_shape=jax.ShapeDtypeStruct((N, D), jnp.float32),
    )(pf, b2f, wfa, degf, pb, b2b, wfb, degb, bfin)


def kernel(x, W1f, b1f, W2f, b2f, W1b, b1b, W2b, b2b, Wfin, bfin, edge_index):
    ei = edge_index.reshape(2, NW, NB, B)
    ones16 = jnp.ones((B, 16), jnp.float32)
    zeros16 = jnp.zeros((N, 16), jnp.float32)
    zeros128 = jnp.zeros((N, D), jnp.float32)
    b1f2 = b1f.reshape(1, D)
    b1b2 = b1b.reshape(1, D)
    b2f2 = b2f.reshape(1, D)
    b2b2 = b2b.reshape(1, D)
    bfin2 = bfin.reshape(1, D)
    wfa = Wfin[:D]
    wfb = Wfin[D:]

    degp = _degrees(ei, ones16, zeros16)
    dinvf = jnp.broadcast_to(
        lax.rsqrt(degp[0, 0, :, 0:1] + degp[0, 1, :, 0:1] + 1.0), (N, D))
    dinvb = jnp.broadcast_to(
        lax.rsqrt(degp[1, 0, :, 0:1] + degp[1, 1, :, 0:1] + 1.0), (N, D))

    z1f, z1b = _tc1(x, W1f, W1b, dinvf, dinvb)
    p1f, p1b = _propagate(ei, z1f, z1b, zeros128)
    z2f, z2b = _tc2(p1f, b1f2, W2f, dinvf, p1b, b1b2, W2b, dinvb)
    p2f, p2b = _propagate(ei, z2f, z2b, zeros128)
    return _tc3(p2f, b2f2, wfa, dinvf, p2b, b2b2, wfb, dinvb, bfin2)


# speedup vs baseline: 1.0150x; 1.0065x over previous
import functools

import jax
import jax.numpy as jnp
from jax import lax
from jax.experimental import pallas as pl
from jax.experimental.pallas import tpu as pltpu
from jax.experimental.pallas import tpu_sc as plsc

N = 10000
E = 320000
D = 128
NC = 2
NS = 16
NW = NC * NS
EW = E // NW
B = 125
NB = EW // B
CH = 40
NCH = NB // CH
RPT = N // NS


def _sc_mesh():
    return plsc.VectorSubcoreMesh(core_axis_name="c", subcore_axis_name="s")


_SC_PARAMS = pltpu.CompilerParams(use_tc_tiling_on_sc=False)


def _deg_kernel(ei_hbm, ones_hbm, z16_hbm, out_hbm, idx_f, idx_b, ones_v,
                acc_f, acc_b):
    c = lax.axis_index("c")
    s = lax.axis_index("s")
    w = s * NC + c
    pltpu.sync_copy(z16_hbm.at[pl.ds(s * RPT, RPT)],
                    acc_f.at[pl.ds(s * RPT, RPT)])
    pltpu.sync_copy(z16_hbm.at[pl.ds(s * RPT, RPT)],
                    acc_b.at[pl.ds(s * RPT, RPT)])
    pltpu.sync_copy(ones_hbm, ones_v)
    pltpu.sync_copy(ei_hbm.at[1, w], idx_f)
    pltpu.sync_copy(ei_hbm.at[0, w], idx_b)
    plsc.subcore_barrier()

    def body(j, carry):
        pltpu.sync_copy(ones_v, acc_f.at[idx_f.at[j]], add=True)
        pltpu.sync_copy(ones_v, acc_b.at[idx_b.at[j]], add=True)
        return carry

    lax.fori_loop(0, NB, body, 0)
    plsc.subcore_barrier()
    pltpu.sync_copy(acc_f.at[pl.ds(s * RPT, RPT)],
                    out_hbm.at[0, c, pl.ds(s * RPT, RPT)])
    pltpu.sync_copy(acc_b.at[pl.ds(s * RPT, RPT)],
                    out_hbm.at[1, c, pl.ds(s * RPT, RPT)])


def _degrees(ei, ones16, zeros16):
    k = pl.kernel(
        _deg_kernel,
        out_type=jax.ShapeDtypeStruct((2, NC, N, 16), jnp.float32),
        mesh=_sc_mesh(),
        compiler_params=_SC_PARAMS,
        scratch_types=[
            pltpu.VMEM((NB, B), jnp.int32),
            pltpu.VMEM((NB, B), jnp.int32),
            pltpu.VMEM((B, 16), jnp.float32),
            pltpu.VMEM_SHARED((N, 16), jnp.float32),
            pltpu.VMEM_SHARED((N, 16), jnp.float32),
        ],
    )
    return k(ei, ones16, zeros16)


def _prop_kernel(ei_hbm, zf_hbm, zb_hbm, z128_hbm, outf_hbm, outb_hbm,
                 gidx, sidx, rows0, rows1, sg0, sg1, ss0, ss1, acc):
    c = lax.axis_index("c")
    s = lax.axis_index("s")
    w = s * NC + c
    for d, z_hbm, out_hbm in ((0, zf_hbm, outf_hbm), (1, zb_hbm, outb_hbm)):
        @pl.when(c == 0)
        def _():
            pltpu.sync_copy(z_hbm.at[pl.ds(s * RPT, RPT)],
                            acc.at[pl.ds(s * RPT, RPT)])

        @pl.when(c == 1)
        def _():
            pltpu.sync_copy(z128_hbm.at[pl.ds(s * RPT, RPT)],
                            acc.at[pl.ds(s * RPT, RPT)])

        plsc.subcore_barrier()

        for ch in range(NCH):
            pltpu.sync_copy(ei_hbm.at[d, w, pl.ds(ch * CH, CH)], gidx)
            pltpu.sync_copy(ei_hbm.at[1 - d, w, pl.ds(ch * CH, CH)], sidx)
            pltpu.async_copy(z_hbm.at[gidx.at[0]], rows0, sg0)

            def body(i, carry):
                b0 = 2 * i
                b1 = b0 + 1
                pltpu.make_async_copy(z_hbm.at[gidx.at[b0]], rows0,
                                      sg0).wait()
                pltpu.async_copy(rows0, acc.at[sidx.at[b0]], ss0, add=True)

                @pl.when(i > 0)
                def _():
                    pltpu.make_async_copy(rows1, acc.at[sidx.at[0]],
                                          ss1).wait()

                pltpu.async_copy(z_hbm.at[gidx.at[b1]], rows1, sg1)
                pltpu.make_async_copy(z_hbm.at[gidx.at[b1]], rows1,
                                      sg1).wait()
                pltpu.async_copy(rows1, acc.at[sidx.at[b1]], ss1, add=True)
                pltpu.make_async_copy(rows0, acc.at[sidx.at[0]], ss0).wait()

                @pl.when(i < CH // 2 - 1)
                def _():
                    pltpu.async_copy(z_hbm.at[gidx.at[b0 + 2]], rows0, sg0)

                return carry

            lax.fori_loop(0, CH // 2, body, 0)
            pltpu.make_async_copy(rows1, acc.at[sidx.at[0]], ss1).wait()
        plsc.subcore_barrier()
        pltpu.sync_copy(acc.at[pl.ds(s * RPT, RPT)],
                        out_hbm.at[c, pl.ds(s * RPT, RPT)])
        plsc.subcore_barrier()


def _propagate(ei, zf, zb, zeros128):
    k = pl.kernel(
        _prop_kernel,
        out_type=(
            jax.ShapeDtypeStruct((NC, N, D), jnp.float32),
            jax.ShapeDtypeStruct((NC, N, D), jnp.float32),
        ),
        mesh=_sc_mesh(),
        compiler_params=_SC_PARAMS,
        scratch_types=[
            pltpu.VMEM((CH, B), jnp.int32),
            pltpu.VMEM((CH, B), jnp.int32),
            pltpu.VMEM((B, D), jnp.float32),
            pltpu.VMEM((B, D), jnp.float32),
            pltpu.SemaphoreType.DMA,
            pltpu.SemaphoreType.DMA,
            pltpu.SemaphoreType.DMA,
            pltpu.SemaphoreType.DMA,
            pltpu.VMEM_SHARED((N, D), jnp.float32),
        ],
    )
    return k(ei, zf, zb, zeros128)


_R = N


def _tc1_body(x_r, w1f_r, w1b_r, df_r, db_r, z1f_r, z1b_r):
    df = df_r[...]
    db = db_r[...]
    xv = x_r[...]
    z1f_r[...] = df * jnp.dot(xv, w1f_r[...], preferred_element_type=jnp.float32)
    z1b_r[...] = db * jnp.dot(xv, w1b_r[...], preferred_element_type=jnp.float32)


def _tc2_body(pf_r, b1f_r, w2f_r, df_r, pb_r, b1b_r, w2b_r,
              db_r, z2f_r, z2b_r):
    df = df_r[...]
    db = db_r[...]
    hf = jnp.maximum(df * (pf_r[0] + pf_r[1]) + b1f_r[...], 0.0)
    hb = jnp.maximum(db * (pb_r[0] + pb_r[1]) + b1b_r[...], 0.0)
    z2f_r[...] = df * jnp.dot(hf, w2f_r[...], preferred_element_type=jnp.float32)
    z2b_r[...] = db * jnp.dot(hb, w2b_r[...], preferred_element_type=jnp.float32)


def _tc3_body(pf_r, b2f_r, wfa_r, df_r, pb_r, b2b_r, wfb_r,
              db_r, bfin_r, out_r):
    df = df_r[...]
    db = db_r[...]
    xf = df * (pf_r[0] + pf_r[1]) + b2f_r[...]
    xb = db * (pb_r[0] + pb_r[1]) + b2b_r[...]
    out_r[...] = (jnp.dot(xf, wfa_r[...], preferred_element_type=jnp.float32)
                  + jnp.dot(xb, wfb_r[...], preferred_element_type=jnp.float32)
                  + bfin_r[...])


def _nd(i):
    return (i, 0)


_SPEC_ROWS = pl.BlockSpec((_R, D), _nd)
_SPEC_W = pl.BlockSpec((D, D), lambda i: (0, 0))
_SPEC_B = pl.BlockSpec((1, D), lambda i: (0, 0))
_SPEC_DINV = pl.BlockSpec((_R, D), lambda i: (i, 0))
_SPEC_P = pl.BlockSpec((NC, _R, D), lambda i: (0, i, 0))
_GRID = (N // _R,)


def _tc1(x, w1f, w1b, degf, degb):
    return pl.pallas_call(
        _tc1_body,
        grid=_GRID,
        in_specs=[_SPEC_ROWS, _SPEC_W, _SPEC_W, _SPEC_DINV, _SPEC_DINV],
        out_specs=[_SPEC_ROWS, _SPEC_ROWS],
        out_shape=[jax.ShapeDtypeStruct((N, D), jnp.float32)] * 2,
    )(x, w1f, w1b, degf, degb)


def _tc2(pf, b1f, w2f, degf, pb, b1b, w2b, degb):
    return pl.pallas_call(
        _tc2_body,
        grid=_GRID,
        in_specs=[_SPEC_P, _SPEC_B, _SPEC_W, _SPEC_DINV,
                  _SPEC_P, _SPEC_B, _SPEC_W, _SPEC_DINV],
        out_specs=[_SPEC_ROWS, _SPEC_ROWS],
        out_shape=[jax.ShapeDtypeStruct((N, D), jnp.float32)] * 2,
    )(pf, b1f, w2f, degf, pb, b1b, w2b, degb)


def _tc3(pf, b2f, wfa, degf, pb, b2b, wfb, degb, bfin):
    return pl.pallas_call(
        _tc3_body,
        grid=_GRID,
        in_specs=[_SPEC_P, _SPEC_B, _SPEC_W, _SPEC_DINV,
                  _SPEC_P, _SPEC_B, _SPEC_W, _SPEC_DINV, _SPEC_B],
        out_specs=_SPEC_ROWS,
        out_shape=jax.ShapeDtypeStruct((N, D), jnp.float32),
    )(pf, b2f, wfa, degf, pb, b2b, wfb, degb, bfin)


def kernel(x, W1f, b1f, W2f, b2f, W1b, b1b, W2b, b2b, Wfin, bfin, edge_index):
    ei = edge_index.reshape(2, NW, NB, B)
    ones16 = jnp.ones((B, 16), jnp.float32)
    zeros16 = jnp.zeros((N, 16), jnp.float32)
    zeros128 = jnp.zeros((N, D), jnp.float32)
    b1f2 = b1f.reshape(1, D)
    b1b2 = b1b.reshape(1, D)
    b2f2 = b2f.reshape(1, D)
    b2b2 = b2b.reshape(1, D)
    bfin2 = bfin.reshape(1, D)
    wfa = Wfin[:D]
    wfb = Wfin[D:]

    degp = _degrees(ei, ones16, zeros16)
    dinvf = jnp.broadcast_to(
        lax.rsqrt(degp[0, 0, :, 0:1] + degp[0, 1, :, 0:1] + 1.0), (N, D))
    dinvb = jnp.broadcast_to(
        lax.rsqrt(degp[1, 0, :, 0:1] + degp[1, 1, :, 0:1] + 1.0), (N, D))

    z1f, z1b = _tc1(x, W1f, W1b, dinvf, dinvb)
    p1f, p1b = _propagate(ei, z1f, z1b, zeros128)
    z2f, z2b = _tc2(p1f, b1f2, W2f, dinvf, p1b, b1b2, W2b, dinvb)
    p2f, p2b = _propagate(ei, z2f, z2b, zeros128)
    return _tc3(p2f, b2f2, wfa, dinvf, p2b, b2b2, wfb, dinvb, bfin2)


# speedup vs baseline: 1.0159x; 1.0009x over previous
import jax
import jax.numpy as jnp
from jax import lax
from jax.experimental import pallas as pl
from jax.experimental.pallas import tpu as pltpu
from jax.experimental.pallas import tpu_sc as plsc

N = 10000
E = 320000
D = 128
NC = 2
NS = 16
NW = NC * NS
EW = E // NW
B = 125
NB = EW // B
CH = 40
NCH = NB // CH
RPT = N // NS


def _sc_mesh():
    return plsc.VectorSubcoreMesh(core_axis_name="c", subcore_axis_name="s")


_SC_PARAMS = pltpu.CompilerParams(use_tc_tiling_on_sc=False)


def _deg_kernel(ei_hbm, ones_hbm, z16_hbm, out_hbm, idx_f, idx_b, ones_v,
                acc_f, acc_b):
    c = lax.axis_index("c")
    s = lax.axis_index("s")
    w = s * NC + c
    pltpu.sync_copy(z16_hbm.at[pl.ds(s * RPT, RPT)],
                    acc_f.at[pl.ds(s * RPT, RPT)])
    pltpu.sync_copy(z16_hbm.at[pl.ds(s * RPT, RPT)],
                    acc_b.at[pl.ds(s * RPT, RPT)])
    pltpu.sync_copy(ones_hbm, ones_v)
    pltpu.sync_copy(ei_hbm.at[1, w], idx_f)
    pltpu.sync_copy(ei_hbm.at[0, w], idx_b)
    plsc.subcore_barrier()

    def body(j, carry):
        pltpu.sync_copy(ones_v, acc_f.at[idx_f.at[j]], add=True)
        pltpu.sync_copy(ones_v, acc_b.at[idx_b.at[j]], add=True)
        return carry

    lax.fori_loop(0, NB, body, 0)
    plsc.subcore_barrier()
    pltpu.sync_copy(acc_f.at[pl.ds(s * RPT, RPT)],
                    out_hbm.at[0, c, pl.ds(s * RPT, RPT)])
    pltpu.sync_copy(acc_b.at[pl.ds(s * RPT, RPT)],
                    out_hbm.at[1, c, pl.ds(s * RPT, RPT)])


def _degrees(ei, ones16, zeros16):
    k = pl.kernel(
        _deg_kernel,
        out_type=jax.ShapeDtypeStruct((2, NC, N, 16), jnp.float32),
        mesh=_sc_mesh(),
        compiler_params=_SC_PARAMS,
        scratch_types=[
            pltpu.VMEM((NB, B), jnp.int32),
            pltpu.VMEM((NB, B), jnp.int32),
            pltpu.VMEM((B, 16), jnp.float32),
            pltpu.VMEM_SHARED((N, 16), jnp.float32),
            pltpu.VMEM_SHARED((N, 16), jnp.float32),
        ],
    )
    return k(ei, ones16, zeros16)


def _prop_kernel(ei_hbm, zf_hbm, zb_hbm, z128_hbm, outf_hbm, outb_hbm,
                 gidx, sidx, rows0, rows1, sg0, sg1, ss0, ss1, acc):
    c = lax.axis_index("c")
    s = lax.axis_index("s")
    w = s * NC + c
    for d, z_hbm, out_hbm in ((0, zf_hbm, outf_hbm), (1, zb_hbm, outb_hbm)):
        @pl.when(c == 0)
        def _():
            pltpu.sync_copy(z_hbm.at[pl.ds(s * RPT, RPT)],
                            acc.at[pl.ds(s * RPT, RPT)])

        @pl.when(c == 1)
        def _():
            pltpu.sync_copy(z128_hbm.at[pl.ds(s * RPT, RPT)],
                            acc.at[pl.ds(s * RPT, RPT)])

        plsc.subcore_barrier()

        for ch in range(NCH):
            pltpu.sync_copy(ei_hbm.at[d, w, pl.ds(ch * CH, CH)], gidx)
            pltpu.sync_copy(ei_hbm.at[1 - d, w, pl.ds(ch * CH, CH)], sidx)
            pltpu.async_copy(z_hbm.at[gidx.at[0]], rows0, sg0)

            def body(i, carry):
                b0 = 2 * i
                b1 = b0 + 1
                pltpu.make_async_copy(z_hbm.at[gidx.at[b0]], rows0,
                                      sg0).wait()
                pltpu.async_copy(rows0, acc.at[sidx.at[b0]], ss0, add=True)

                @pl.when(i > 0)
                def _():
                    pltpu.make_async_copy(rows1, acc.at[sidx.at[0]],
                                          ss1).wait()

                pltpu.async_copy(z_hbm.at[gidx.at[b1]], rows1, sg1)
                pltpu.make_async_copy(z_hbm.at[gidx.at[b1]], rows1,
                                      sg1).wait()
                pltpu.async_copy(rows1, acc.at[sidx.at[b1]], ss1, add=True)
                pltpu.make_async_copy(rows0, acc.at[sidx.at[0]], ss0).wait()

                @pl.when(i < CH // 2 - 1)
                def _():
                    pltpu.async_copy(z_hbm.at[gidx.at[b0 + 2]], rows0, sg0)

                return carry

            lax.fori_loop(0, CH // 2, body, 0)
            pltpu.make_async_copy(rows1, acc.at[sidx.at[0]], ss1).wait()
        plsc.subcore_barrier()
        pltpu.sync_copy(acc.at[pl.ds(s * RPT, RPT)],
                        out_hbm.at[c, pl.ds(s * RPT, RPT)])
        plsc.subcore_barrier()


def _propagate(ei, zf, zb, zeros128):
    k = pl.kernel(
        _prop_kernel,
        out_type=(
            jax.ShapeDtypeStruct((NC, N, D), jnp.float32),
            jax.ShapeDtypeStruct((NC, N, D), jnp.float32),
        ),
        mesh=_sc_mesh(),
        compiler_params=_SC_PARAMS,
        scratch_types=[
            pltpu.VMEM((CH, B), jnp.int32),
            pltpu.VMEM((CH, B), jnp.int32),
            pltpu.VMEM((B, D), jnp.float32),
            pltpu.VMEM((B, D), jnp.float32),
            pltpu.SemaphoreType.DMA,
            pltpu.SemaphoreType.DMA,
            pltpu.SemaphoreType.DMA,
            pltpu.SemaphoreType.DMA,
            pltpu.VMEM_SHARED((N, D), jnp.float32),
        ],
    )
    return k(ei, zf, zb, zeros128)


_R = N


def _tc1_body(x_r, w1f_r, w1b_r, df_r, db_r, z1f_r, z1b_r):
    df = df_r[...]
    db = db_r[...]
    xv = x_r[...]
    z1f_r[...] = df * jnp.dot(xv, w1f_r[...], preferred_element_type=jnp.float32)
    z1b_r[...] = db * jnp.dot(xv, w1b_r[...], preferred_element_type=jnp.float32)


def _tc2_body(pf_r, b1f_r, w2f_r, df_r, pb_r, b1b_r, w2b_r,
              db_r, z2f_r, z2b_r):
    df = df_r[...]
    db = db_r[...]
    hf = jnp.maximum(df * (pf_r[0] + pf_r[1]) + b1f_r[...], 0.0)
    hb = jnp.maximum(db * (pb_r[0] + pb_r[1]) + b1b_r[...], 0.0)
    z2f_r[...] = df * jnp.dot(hf, w2f_r[...], preferred_element_type=jnp.float32)
    z2b_r[...] = db * jnp.dot(hb, w2b_r[...], preferred_element_type=jnp.float32)


def _tc3_body(pf_r, b2f_r, wfa_r, df_r, pb_r, b2b_r, wfb_r,
              db_r, bfin_r, out_r):
    df = df_r[...]
    db = db_r[...]
    xf = df * (pf_r[0] + pf_r[1]) + b2f_r[...]
    xb = db * (pb_r[0] + pb_r[1]) + b2b_r[...]
    out_r[...] = (jnp.dot(xf, wfa_r[...], preferred_element_type=jnp.float32)
                  + jnp.dot(xb, wfb_r[...], preferred_element_type=jnp.float32)
                  + bfin_r[...])


def _nd(i):
    return (i, 0)


_SPEC_ROWS = pl.BlockSpec((_R, D), _nd)
_SPEC_W = pl.BlockSpec((D, D), lambda i: (0, 0))
_SPEC_B = pl.BlockSpec((1, D), lambda i: (0, 0))
_SPEC_DINV = pl.BlockSpec((_R, D), lambda i: (i, 0))
_SPEC_P = pl.BlockSpec((NC, _R, D), lambda i: (0, i, 0))
_GRID = (N // _R,)


def _tc1(x, w1f, w1b, degf, degb):
    return pl.pallas_call(
        _tc1_body,
        grid=_GRID,
        in_specs=[_SPEC_ROWS, _SPEC_W, _SPEC_W, _SPEC_DINV, _SPEC_DINV],
        out_specs=[_SPEC_ROWS, _SPEC_ROWS],
        out_shape=[jax.ShapeDtypeStruct((N, D), jnp.float32)] * 2,
    )(x, w1f, w1b, degf, degb)


def _tc2(pf, b1f, w2f, degf, pb, b1b, w2b, degb):
    return pl.pallas_call(
        _tc2_body,
        grid=_GRID,
        in_specs=[_SPEC_P, _SPEC_B, _SPEC_W, _SPEC_DINV,
                  _SPEC_P, _SPEC_B, _SPEC_W, _SPEC_DINV],
        out_specs=[_SPEC_ROWS, _SPEC_ROWS],
        out_shape=[jax.ShapeDtypeStruct((N, D), jnp.float32)] * 2,
    )(pf, b1f, w2f, degf, pb, b1b, w2b, degb)


def _tc3(pf, b2f, wfa, degf, pb, b2b, wfb, degb, bfin):
    return pl.pallas_call(
        _tc3_body,
        grid=_GRID,
        in_specs=[_SPEC_P, _SPEC_B, _SPEC_W, _SPEC_DINV,
                  _SPEC_P, _SPEC_B, _SPEC_W, _SPEC_DINV, _SPEC_B],
        out_specs=_SPEC_ROWS,
        out_shape=jax.ShapeDtypeStruct((N, D), jnp.float32),
    )(pf, b2f, wfa, degf, pb, b2b, wfb, degb, bfin)


def kernel(x, W1f, b1f, W2f, b2f, W1b, b1b, W2b, b2b, Wfin, bfin, edge_index):
    ei = edge_index.reshape(2, NW, NB, B)
    ones16 = jnp.ones((B, 16), jnp.float32)
    zeros16 = jnp.zeros((N, 16), jnp.float32)
    zeros128 = jnp.zeros((N, D), jnp.float32)
    b1f2 = b1f.reshape(1, D)
    b1b2 = b1b.reshape(1, D)
    b2f2 = b2f.reshape(1, D)
    b2b2 = b2b.reshape(1, D)
    bfin2 = bfin.reshape(1, D)
    wfa = Wfin[:D]
    wfb = Wfin[D:]

    degp = _degrees(ei, ones16, zeros16)
    dinvf = jnp.broadcast_to(
        lax.rsqrt(degp[0, 0, :, 0:1] + degp[0, 1, :, 0:1] + 1.0), (N, D))
    dinvb = jnp.broadcast_to(
        lax.rsqrt(degp[1, 0, :, 0:1] + degp[1, 1, :, 0:1] + 1.0), (N, D))

    z1f, z1b = _tc1(x, W1f, W1b, dinvf, dinvb)
    p1f, p1b = _propagate(ei, z1f, z1b, zeros128)
    z2f, z2b = _tc2(p1f, b1f2, W2f, dinvf, p1b, b1b2, W2b, dinvb)
    p2f, p2b = _propagate(ei, z2f, z2b, zeros128)
    return _tc3(p2f, b2f2, wfa, dinvf, p2b, b2b2, wfb, dinvb, bfin2)
